# baseline (speedup 1.0000x reference)
import functools

import jax
import jax.numpy as jnp
from jax import lax
from jax.experimental import pallas as pl
from jax.experimental.pallas import tpu as pltpu
from jax.experimental.pallas import tpu_sc as plsc

D = 32
B = 16384
L = 50
NC = 2
NS = 16
NW = NC * NS
BW = B // NW
CL = 128
CH = BW // CL
GC = 64
GN = BW // GC

_mesh = plsc.VectorSubcoreMesh(core_axis_name="c", subcore_axis_name="s")


@functools.partial(
    pl.kernel,
    mesh=_mesh,
    out_type=jax.ShapeDtypeStruct((B,), jnp.float32),
    compiler_params=pltpu.CompilerParams(
        needs_layout_passes=False, use_tc_tiling_on_sc=False),
    scratch_types=[
        pltpu.VMEM((L * CH, CL), jnp.int32),
        pltpu.VMEM((BW,), jnp.int32),
        pltpu.VMEM((BW,), jnp.int32),
        pltpu.VMEM((BW,), jnp.int32),
        pltpu.VMEM((BW,), jnp.int32),
        pltpu.VMEM((BW,), jnp.float32),
        pltpu.VMEM((GC, 128), jnp.float32),
        pltpu.VMEM((GC, 128), jnp.float32),
        pltpu.VMEM((GC, 128), jnp.float32),
        pltpu.VMEM((D, BW), jnp.float32),
        pltpu.VMEM((BW, D), jnp.float32),
        pltpu.VMEM((CL, D), jnp.float32),
        pltpu.VMEM((BW,), jnp.float32),
        pltpu.SemaphoreType.DMA,
    ],
)
def _fpmc_sc(seq_idx_hbm, user_hbm, item_hbm, seqlen_hbm,
             wui_hbm, wiu_hbm, wli_hbm, wil_hbm, out_hbm,
             seq_idx_v, user_v, item_v, gu_v, gi_v, seqlen_v,
             bufu_v, bufi_v, bufl_v, vilt_v, acc_v, dummy_v, out_v, sem0):
    wid = lax.axis_index("s") * NC + lax.axis_index("c")
    base = wid * BW

    pltpu.sync_copy(seq_idx_hbm.at[wid], seq_idx_v)
    pltpu.sync_copy(user_hbm.at[wid], user_v)
    pltpu.sync_copy(item_hbm.at[wid], item_v)
    pltpu.sync_copy(seqlen_hbm.at[wid], seqlen_v)

    zero = jnp.zeros((16,), jnp.float32)
    iota16 = lax.iota(jnp.int32, 16)

    def gidx(i, carry):
        off = pl.multiple_of(i * 16, 16)
        u = user_v[pl.ds(off, 16)]
        it = item_v[pl.ds(off, 16)]
        gu_v[pl.ds(off, 16)] = lax.shift_right_logical(u, 2)
        gi_v[pl.ds(off, 16)] = lax.shift_right_logical(it, 2)
        return carry
    lax.fori_loop(0, BW // 16, gidx, 0)

    def zbody(i, carry):
        acc_v[i, pl.ds(0, 16)] = zero
        acc_v[i, pl.ds(16, 16)] = zero
        return carry
    lax.fori_loop(0, BW, zbody, 0)

    def p1(c, carry):
        coff = pl.multiple_of(c * GC, GC)
        pltpu.async_copy(wui_hbm.at[gu_v.at[pl.ds(coff, GC)]], bufu_v, sem0)
        pltpu.async_copy(wiu_hbm.at[gi_v.at[pl.ds(coff, GC)]], bufi_v, sem0)
        pltpu.async_copy(wil_hbm.at[gi_v.at[pl.ds(coff, GC)]], bufl_v, sem0)
        pltpu.make_async_copy(wui_hbm.at[pl.ds(0, GC)], bufu_v, sem0).wait()
        pltpu.make_async_copy(wui_hbm.at[pl.ds(0, GC)], bufi_v, sem0).wait()
        pltpu.make_async_copy(wui_hbm.at[pl.ds(0, GC)], bufl_v, sem0).wait()

        def jgrp(j, carry2):
            joff = pl.multiple_of(j * 16, 16)
            boff = coff + joff
            u = user_v[pl.ds(boff, 16)]
            it = item_v[pl.ds(boff, 16)]
            cu = lax.shift_left(jnp.bitwise_and(u, 3), 5)
            ci = lax.shift_left(jnp.bitwise_and(it, 3), 5)
            rows = joff + iota16
            a1 = zero
            for d in range(D):
                vu = plsc.load_gather(bufu_v, [rows, cu + d])
                vi = plsc.load_gather(bufi_v, [rows, ci + d])
                vl = plsc.load_gather(bufl_v, [rows, ci + d])
                a1 = a1 + vu * vi
                vilt_v[d, pl.ds(boff, 16)] = vl
            out_v[pl.ds(boff, 16)] = a1
            return carry2
        lax.fori_loop(0, GC // 16, jgrp, 0)
        return carry
    lax.fori_loop(0, GN, p1, 0)

    def fire(t, carry):
        c = lax.rem(t, CH)
        pltpu.async_copy(wli_hbm.at[seq_idx_v.at[t]],
                         acc_v.at[pl.ds(c * CL, CL)], sem0, add=True)
        return carry
    lax.fori_loop(0, L * CH, fire, 0)

    def drain(t, carry):
        pltpu.make_async_copy(wli_hbm.at[pl.ds(0, CL)], dummy_v, sem0).wait()
        return carry
    lax.fori_loop(0, L * CH, drain, 0)

    def gbody(g, carry):
        goff = pl.multiple_of(g * 16, 16)
        rows = goff + iota16
        a2 = zero
        for d in range(D):
            cols = jnp.full((16,), d, jnp.int32)
            sacc = plsc.load_gather(acc_v, [rows, cols])
            a2 = a2 + sacc * vilt_v[d, pl.ds(goff, 16)]
        sl = seqlen_v[pl.ds(goff, 16)]
        out_v[pl.ds(goff, 16)] = out_v[pl.ds(goff, 16)] + a2 / sl
        return carry
    lax.fori_loop(0, BW // 16, gbody, 0)

    pltpu.sync_copy(out_v, out_hbm.at[pl.ds(base, BW)])


def kernel(user, item, item_seq, seq_len, W_UI, W_IU, W_LI, W_IL):
    user_r = user.astype(jnp.int32).reshape(NW, BW)
    item_r = item.astype(jnp.int32).reshape(NW, BW)
    seq_r = (item_seq.astype(jnp.int32)
             .reshape(NW, CH, CL, L).transpose(0, 3, 1, 2).reshape(NW, L * CH, CL))
    sl_r = seq_len.reshape(NW, BW)
    wui_g = W_UI.reshape(-1, 128)
    wiu_g = W_IU.reshape(-1, 128)
    wil_g = W_IL.reshape(-1, 128)
    return _fpmc_sc(seq_r, user_r, item_r, sl_r, wui_g, wiu_g, W_LI, wil_g)


# speedup vs baseline: 1.0055x; 1.0055x over previous
import functools

import jax
import jax.numpy as jnp
from jax import lax
from jax.experimental import pallas as pl
from jax.experimental.pallas import tpu as pltpu
from jax.experimental.pallas import tpu_sc as plsc

D = 32
B = 16384
L = 50
NC = 2
NS = 16
NW = NC * NS
BW = B // NW
CL = 128
CH = BW // CL
GC = 64
GN = BW // GC
AR = BW * D // 128

_mesh = plsc.VectorSubcoreMesh(core_axis_name="c", subcore_axis_name="s")


@functools.partial(
    pl.kernel,
    mesh=_mesh,
    out_type=jax.ShapeDtypeStruct((B * D // 128, 128), jnp.float32),
    compiler_params=pltpu.CompilerParams(
        needs_layout_passes=False, use_tc_tiling_on_sc=False),
    scratch_types=[
        pltpu.VMEM((L * CH, CL), jnp.int32),
        pltpu.VMEM((BW, D), jnp.float32),
        pltpu.VMEM((AR, 128), jnp.float32),
        pltpu.VMEM((CL, D), jnp.float32),
        pltpu.SemaphoreType.DMA,
    ],
)
def _fpmc_seqsum(seq_idx_hbm, wli_hbm, acc_hbm,
                 seq_idx_v, acc_v, accb_v, dummy_v, sem0):
    wid = lax.axis_index("s") * NC + lax.axis_index("c")

    pltpu.sync_copy(seq_idx_hbm.at[wid], seq_idx_v)

    zero = jnp.zeros((16,), jnp.float32)

    def zbody(i, carry):
        acc_v[i, pl.ds(0, 16)] = zero
        acc_v[i, pl.ds(16, 16)] = zero
        return carry
    lax.fori_loop(0, BW, zbody, 0)

    def fire(t, carry):
        c = lax.rem(t, CH)
        pltpu.async_copy(wli_hbm.at[seq_idx_v.at[t]],
                         acc_v.at[pl.ds(c * CL, CL)], sem0, add=True)
        return carry
    lax.fori_loop(0, L * CH, fire, 0)

    def drain(t, carry):
        pltpu.make_async_copy(wli_hbm.at[pl.ds(0, CL)], dummy_v, sem0).wait()
        return carry
    lax.fori_loop(0, L * CH, drain, 0)

    def rbody(r, carry):
        for j in range(8):
            accb_v[r, pl.ds(j * 16, 16)] = (
                acc_v[4 * r + j // 2, pl.ds((j % 2) * 16, 16)])
        return carry
    lax.fori_loop(0, AR, rbody, 0)

    pltpu.sync_copy(accb_v, acc_hbm.at[pl.ds(wid * AR, AR)])


@functools.partial(
    pl.kernel,
    mesh=_mesh,
    out_type=jax.ShapeDtypeStruct((B,), jnp.float32),
    compiler_params=pltpu.CompilerParams(
        needs_layout_passes=False, use_tc_tiling_on_sc=True),
    scratch_types=[
        pltpu.VMEM((BW,), jnp.int32),
        pltpu.VMEM((BW,), jnp.int32),
        pltpu.VMEM((BW,), jnp.int32),
        pltpu.VMEM((BW,), jnp.int32),
        pltpu.VMEM((BW,), jnp.float32),
        pltpu.VMEM((GC, 128), jnp.float32),
        pltpu.VMEM((GC, 128), jnp.float32),
        pltpu.VMEM((GC, 128), jnp.float32),
        pltpu.VMEM((D, BW), jnp.float32),
        pltpu.VMEM((AR, 128), jnp.float32),
        pltpu.VMEM((BW,), jnp.float32),
        pltpu.SemaphoreType.DMA,
    ],
)
def _fpmc_dots(user_hbm, item_hbm, seqlen_hbm,
               wui_hbm, wiu_hbm, wil_hbm, acc_hbm, out_hbm,
               user_v, item_v, gu_v, gi_v, seqlen_v,
               bufu_v, bufi_v, bufl_v, vilt_v, accb_v, out_v, sem0):
    wid = lax.axis_index("s") * NC + lax.axis_index("c")
    base = wid * BW

    pltpu.sync_copy(user_hbm.at[pl.ds(base, BW)], user_v)
    pltpu.sync_copy(item_hbm.at[pl.ds(base, BW)], item_v)
    pltpu.sync_copy(seqlen_hbm.at[pl.ds(base, BW)], seqlen_v)
    pltpu.sync_copy(acc_hbm.at[pl.ds(wid * AR, AR)], accb_v)

    zero = jnp.zeros((16,), jnp.float32)
    iota16 = lax.iota(jnp.int32, 16)

    def gidx(i, carry):
        off = pl.multiple_of(i * 16, 16)
        u = user_v[pl.ds(off, 16)]
        it = item_v[pl.ds(off, 16)]
        gu_v[pl.ds(off, 16)] = lax.shift_right_logical(u, 2)
        gi_v[pl.ds(off, 16)] = lax.shift_right_logical(it, 2)
        return carry
    lax.fori_loop(0, BW // 16, gidx, 0)

    def p1(c, carry):
        coff = pl.multiple_of(c * GC, GC)
        pltpu.async_copy(wui_hbm.at[gu_v.at[pl.ds(coff, GC)]], bufu_v, sem0)
        pltpu.async_copy(wiu_hbm.at[gi_v.at[pl.ds(coff, GC)]], bufi_v, sem0)
        pltpu.async_copy(wil_hbm.at[gi_v.at[pl.ds(coff, GC)]], bufl_v, sem0)
        pltpu.make_async_copy(wui_hbm.at[pl.ds(0, GC)], bufu_v, sem0).wait()
        pltpu.make_async_copy(wui_hbm.at[pl.ds(0, GC)], bufi_v, sem0).wait()
        pltpu.make_async_copy(wui_hbm.at[pl.ds(0, GC)], bufl_v, sem0).wait()

        def jgrp(j, carry2):
            joff = pl.multiple_of(j * 16, 16)
            boff = coff + joff
            u = user_v[pl.ds(boff, 16)]
            it = item_v[pl.ds(boff, 16)]
            cu = lax.shift_left(jnp.bitwise_and(u, 3), 5)
            ci = lax.shift_left(jnp.bitwise_and(it, 3), 5)
            rows = joff + iota16
            a1 = zero
            for d in range(D):
                vu = plsc.load_gather(bufu_v, [rows, cu + d])
                vi = plsc.load_gather(bufi_v, [rows, ci + d])
                vl = plsc.load_gather(bufl_v, [rows, ci + d])
                a1 = a1 + vu * vi
                vilt_v[d, pl.ds(boff, 16)] = vl
            out_v[pl.ds(boff, 16)] = a1
            return carry2
        lax.fori_loop(0, GC // 16, jgrp, 0)
        return carry
    lax.fori_loop(0, GN, p1, 0)

    def gbody(g, carry):
        goff = pl.multiple_of(g * 16, 16)
        e_base = lax.shift_left(goff + iota16, 5)
        a2 = zero
        for d in range(D):
            e = e_base + d
            rows = lax.shift_right_logical(e, 7)
            cols = jnp.bitwise_and(e, 127)
            sacc = plsc.load_gather(accb_v, [rows, cols])
            a2 = a2 + sacc * vilt_v[d, pl.ds(goff, 16)]
        sl = seqlen_v[pl.ds(goff, 16)]
        out_v[pl.ds(goff, 16)] = out_v[pl.ds(goff, 16)] + a2 / sl
        return carry
    lax.fori_loop(0, BW // 16, gbody, 0)

    pltpu.sync_copy(out_v, out_hbm.at[pl.ds(base, BW)])


def kernel(user, item, item_seq, seq_len, W_UI, W_IU, W_LI, W_IL):
    user_i = user.astype(jnp.int32)
    item_i = item.astype(jnp.int32)
    seq_r = (item_seq.astype(jnp.int32)
             .reshape(NW, CH, CL, L).transpose(0, 3, 1, 2).reshape(NW, L * CH, CL))
    wui_g = W_UI.reshape(-1, 128)
    wiu_g = W_IU.reshape(-1, 128)
    wil_g = W_IL.reshape(-1, 128)
    acc = _fpmc_seqsum(seq_r, W_LI)
    return _fpmc_dots(user_i, item_i, seq_len, wui_g, wiu_g, wil_g, acc)
